# Initial kernel scaffold; baseline (speedup 1.0000x reference)
import jax
import jax.numpy as jnp
from jax import lax
from jax.experimental import pallas as pl
from jax.experimental.pallas import tpu as pltpu
from jax.experimental.pallas import tpu_sc as plsc

N = 50000
E = 800000
D = 64
HD = D // 2

NC, NS = 2, 16
NW = NC * NS

ACC_ROWS = 51200
DUMP = N
EPAD = 16 * ACC_ROWS
EW = EPAD // NS
HW = EPAD // NW
CH = 1024
NCHUNK_CONV = EW // CH
NCHUNK_HIST = HW // CH

BN = 2000
GRID_N = N // BN

_mesh = plsc.VectorSubcoreMesh(core_axis_name="c", subcore_axis_name="s")


def _hist_body(dst_hbm, hist_hbm, dv, hist_v):
    k = lax.axis_index("c")
    s = lax.axis_index("s")
    wid = k * NS + s

    @pl.loop(0, ACC_ROWS, step=16)
    def _(i):
        hist_v[pl.ds(i, 16)] = jnp.zeros((16,), jnp.float32)

    base = wid * HW

    @pl.loop(0, NCHUNK_HIST)
    def _(c):
        pltpu.sync_copy(dst_hbm.at[pl.ds(base + c * CH, CH)], dv)

        @pl.loop(0, CH, step=16)
        def _(i):
            plsc.addupdate_scatter(hist_v, [dv[pl.ds(i, 16)]],
                                   jnp.ones((16,), jnp.float32))

    pltpu.sync_copy(hist_v, hist_hbm.at[wid])


@jax.jit
def _sc_hist(dstp):
    kern = pl.kernel(
        _hist_body,
        out_type=jax.ShapeDtypeStruct((NW, ACC_ROWS), jnp.float32),
        mesh=_mesh,
        scratch_types=[
            pltpu.VMEM((CH,), jnp.int32),
            pltpu.VMEM((ACC_ROWS,), jnp.float32),
        ],
    )
    return kern(dstp)


def _conv_body(y_hbm, src_hbm, dst_hbm, zero_hbm, acc_hbm,
               sv, gv, dv, rows_v, zb_v, gsem, ssem, acc_sh):
    k = lax.axis_index("c")
    s = lax.axis_index("s")

    pltpu.sync_copy(zero_hbm, zb_v)

    @pl.loop(0, 25)
    def _(i):
        pltpu.sync_copy(zb_v, acc_sh.at[pl.ds((s * 25 + i) * 128, 128)])

    plsc.subcore_barrier()

    base = s * EW

    @pl.loop(0, NCHUNK_CONV)
    def _(c):
        eb = base + c * CH
        pltpu.sync_copy(src_hbm.at[pl.ds(eb, CH)], sv)
        pltpu.sync_copy(dst_hbm.at[pl.ds(eb // 128, CH // 128)], dv)

        @pl.loop(0, CH, step=16)
        def _(i):
            gv[pl.ds(i, 16)] = sv[pl.ds(i, 16)] * 2 + k

        cps = [
            pltpu.async_copy(
                y_hbm.at[gv.at[pl.ds(j * 128, 128)]],
                rows_v.at[pl.ds(j * 128, 128)], gsem)
            for j in range(CH // 128)
        ]
        for cp in cps:
            cp.wait()
        cps = [
            pltpu.async_copy(
                rows_v.at[pl.ds(j * 128, 128)],
                acc_sh.at[dv.at[j]], ssem, add=True)
            for j in range(CH // 128)
        ]
        for cp in cps:
            cp.wait()

    plsc.subcore_barrier()
    pltpu.sync_copy(acc_sh.at[pl.ds(s * (ACC_ROWS // NS), ACC_ROWS // NS)],
                    acc_hbm.at[k].at[pl.ds(s * (ACC_ROWS // NS),
                                           ACC_ROWS // NS)])


@jax.jit
def _sc_conv(yv, srcp, dst2, zrows):
    kern = pl.kernel(
        _conv_body,
        out_type=jax.ShapeDtypeStruct((NC, ACC_ROWS, HD), jnp.float32),
        mesh=_mesh,
        scratch_types=[
            pltpu.VMEM((CH,), jnp.int32),
            pltpu.VMEM((CH,), jnp.int32),
            pltpu.VMEM((CH // 128, 128), jnp.int32),
            pltpu.VMEM((CH, HD), jnp.float32),
            pltpu.VMEM((128, HD), jnp.float32),
            pltpu.SemaphoreType.DMA,
            pltpu.SemaphoreType.DMA,
            pltpu.VMEM_SHARED((ACC_ROWS, HD), jnp.float32),
        ],
    )
    return kern(yv, srcp, dst2, zrows)


def _prep_body(hist_ref, x_ref, y_ref, d_ref):
    h = hist_ref[...]
    deg = jnp.sum(h, axis=0, keepdims=True)
    dinv = jnp.where(deg > 0, lax.rsqrt(deg), 0.0)
    dcol = jnp.transpose(dinv, (1, 0))
    d_ref[...] = dcol
    y_ref[...] = x_ref[...] * dcol


@jax.jit
def _tc_prep(hist, x):
    return pl.pallas_call(
        _prep_body,
        grid=(GRID_N,),
        in_specs=[
            pl.BlockSpec((NW, BN), lambda i: (0, i)),
            pl.BlockSpec((BN, D), lambda i: (i, 0)),
        ],
        out_specs=[
            pl.BlockSpec((BN, D), lambda i: (i, 0)),
            pl.BlockSpec((BN, 1), lambda i: (i, 0)),
        ],
        out_shape=[
            jax.ShapeDtypeStruct((N, D), jnp.float32),
            jax.ShapeDtypeStruct((N, 1), jnp.float32),
        ],
    )(hist, x)


def _mid_body(acc_ref, d_ref, y_ref):
    a = acc_ref[...]
    merged = jnp.concatenate([a[0], a[1]], axis=1)
    d = d_ref[...]
    y_ref[...] = merged * (d * d)


@jax.jit
def _tc_mid(acc, d):
    return pl.pallas_call(
        _mid_body,
        grid=(GRID_N,),
        in_specs=[
            pl.BlockSpec((NC, BN, HD), lambda i: (0, i, 0)),
            pl.BlockSpec((BN, 1), lambda i: (i, 0)),
        ],
        out_specs=pl.BlockSpec((BN, D), lambda i: (i, 0)),
        out_shape=jax.ShapeDtypeStruct((N, D), jnp.float32),
    )(acc, d)


def _final_body(x_ref, a1_ref, a2_ref, d_ref, o_ref):
    a1 = a1_ref[...]
    a2 = a2_ref[...]
    e1 = jnp.concatenate([a1[0], a1[1]], axis=1)
    e2 = jnp.concatenate([a2[0], a2[1]], axis=1)
    d = d_ref[...]
    o_ref[...] = (x_ref[...] + d * e1 + d * e2) * (1.0 / 3.0)


@jax.jit
def _tc_final(x, acc1, acc2, d):
    return pl.pallas_call(
        _final_body,
        grid=(GRID_N,),
        in_specs=[
            pl.BlockSpec((BN, D), lambda i: (i, 0)),
            pl.BlockSpec((NC, BN, HD), lambda i: (0, i, 0)),
            pl.BlockSpec((NC, BN, HD), lambda i: (0, i, 0)),
            pl.BlockSpec((BN, 1), lambda i: (i, 0)),
        ],
        out_specs=pl.BlockSpec((BN, D), lambda i: (i, 0)),
        out_shape=jax.ShapeDtypeStruct((N, D), jnp.float32),
    )(x, acc1, acc2, d)


def kernel(x, edge_index):
    src = edge_index[0].astype(jnp.int32)
    dst = edge_index[1].astype(jnp.int32)
    srcp = jnp.concatenate([src, jnp.zeros((EPAD - E,), jnp.int32)])
    dstp = jnp.concatenate([dst, jnp.full((EPAD - E,), DUMP, jnp.int32)])
    dst2 = dstp.reshape(EPAD // 128, 128)
    zrows = jnp.zeros((128, HD), jnp.float32)

    hist = _sc_hist(dstp)
    y1, d = _tc_prep(hist[:, :N], x)
    acc1 = _sc_conv(y1.reshape(2 * N, HD), srcp, dst2, zrows)
    y2 = _tc_mid(acc1[:, :N, :], d)
    acc2 = _sc_conv(y2.reshape(2 * N, HD), srcp, dst2, zrows)
    return _tc_final(x, acc1[:, :N, :], acc2[:, :N, :], d)


# speedup vs baseline: 12.6422x; 12.6422x over previous
import jax
import jax.numpy as jnp
from jax import lax
from jax.experimental import pallas as pl
from jax.experimental.pallas import tpu as pltpu
from jax.experimental.pallas import tpu_sc as plsc

N = 50000
E = 800000
D = 64
HD = D // 2

NC, NS = 2, 16
NW = NC * NS

ACC_ROWS = 51200
DUMP = N
EPAD = 16 * ACC_ROWS
EW = EPAD // NS
HW = EPAD // NW
CH = 1024
CCH = 512
NCHUNK_CONV = EW // CCH
NCHUNK_HIST = HW // CH

NP = ACC_ROWS
BN = 2048
GRID_N = NP // BN

import functools


@functools.lru_cache(maxsize=1)
def _mesh():
    return plsc.VectorSubcoreMesh(core_axis_name="c", subcore_axis_name="s")


_SC_PARAMS = pltpu.CompilerParams(needs_layout_passes=False,
                                 use_tc_tiling_on_sc=False)


def _hist_body(dst_hbm, hist_hbm, dv, hist_v):
    k = lax.axis_index("c")
    s = lax.axis_index("s")
    wid = k * NS + s

    @pl.loop(0, ACC_ROWS, step=16)
    def _(i):
        hist_v[pl.ds(i, 16)] = jnp.zeros((16,), jnp.float32)

    base = wid * HW

    @pl.loop(0, NCHUNK_HIST)
    def _(c):
        pltpu.sync_copy(dst_hbm.at[pl.ds(base + c * CH, CH)], dv)

        @pl.loop(0, CH, step=16)
        def _(i):
            plsc.addupdate_scatter(hist_v, [dv[pl.ds(i, 16)]],
                                   jnp.ones((16,), jnp.float32))

    pltpu.sync_copy(hist_v, hist_hbm.at[wid])


@jax.jit
def _sc_hist(dstp):
    kern = pl.kernel(
        _hist_body,
        out_type=jax.ShapeDtypeStruct((NW, ACC_ROWS), jnp.float32),
        mesh=_mesh(),
        scratch_types=[
            pltpu.VMEM((CH,), jnp.int32),
            pltpu.VMEM((ACC_ROWS,), jnp.float32),
        ],
        compiler_params=_SC_PARAMS,
    )
    return kern(dstp)


def _conv_body(y_hbm, src_hbm, dst_hbm, zero_hbm, acc_hbm,
               sv, gv, dsv, dv, rows_v, zb_v, gsem, acc_sh):
    k = lax.axis_index("c")
    s = lax.axis_index("s")

    pltpu.sync_copy(zero_hbm, zb_v)

    @pl.loop(0, 25)
    def _(i):
        pltpu.sync_copy(zb_v, acc_sh.at[pl.ds((s * 25 + i) * 128, 128)])

    plsc.subcore_barrier()

    base = s * EW

    @pl.loop(0, NCHUNK_CONV)
    def _(c):
        eb = base + c * CCH
        pltpu.sync_copy(src_hbm.at[pl.ds(eb, CCH)], sv)
        pltpu.sync_copy(dst_hbm.at[pl.ds(eb, CCH)], dsv)

        @pl.loop(0, CCH, step=16)
        def _(i):
            gv[pl.ds(i, 16)] = sv[pl.ds(i, 16)] * 2 + k

        for r in range(CCH // 128):
            @pl.loop(0, 128, step=16)
            def _(i):
                dv[r, pl.ds(i, 16)] = dsv[pl.ds(r * 128 + i, 16)]

        cps = [
            pltpu.async_copy(
                y_hbm.at[gv.at[pl.ds(j * 128, 128)]],
                rows_v.at[pl.ds(j * 128, 128)], gsem)
            for j in range(CCH // 128)
        ]
        for cp in cps:
            cp.wait()
        for j in range(CCH // 128):
            pltpu.sync_copy(rows_v.at[pl.ds(j * 128, 128)],
                            acc_sh.at[dv.at[j]], add=True)

    plsc.subcore_barrier()

    @pl.loop(0, 25)
    def _(i):
        off = (s * 25 + i) * 128
        pltpu.sync_copy(acc_sh.at[pl.ds(off, 128)], zb_v)
        pltpu.sync_copy(zb_v, acc_hbm.at[k].at[pl.ds(off, 128)])


@jax.jit
def _sc_conv(yv, srcp, dstp, zrows):
    kern = pl.kernel(
        _conv_body,
        out_type=jax.ShapeDtypeStruct((NC, ACC_ROWS, HD), jnp.float32),
        mesh=_mesh(),
        scratch_types=[
            pltpu.VMEM((CCH,), jnp.int32),
            pltpu.VMEM((CCH,), jnp.int32),
            pltpu.VMEM((CCH,), jnp.int32),
            pltpu.VMEM((CCH // 128, 128), jnp.int32),
            pltpu.VMEM((CCH, HD), jnp.float32),
            pltpu.VMEM((128, HD), jnp.float32),
            pltpu.SemaphoreType.DMA,
            pltpu.VMEM_SHARED((ACC_ROWS, HD), jnp.float32),
        ],
        compiler_params=_SC_PARAMS,
    )
    return kern(yv, srcp, dstp, zrows)


def _prep_body(hist_ref, x_ref, y_ref, d_ref):
    h = hist_ref[...]
    deg = jnp.sum(h, axis=0, keepdims=True)
    dinv = jnp.where(deg > 0, lax.rsqrt(deg), 0.0)
    dcol = jnp.transpose(dinv, (1, 0))
    d_ref[...] = dcol
    y_ref[...] = x_ref[...] * dcol


@jax.jit
def _tc_prep(hist, x):
    return pl.pallas_call(
        _prep_body,
        grid=(GRID_N,),
        in_specs=[
            pl.BlockSpec((NW, BN), lambda i: (0, i)),
            pl.BlockSpec((BN, D), lambda i: (i, 0)),
        ],
        out_specs=[
            pl.BlockSpec((BN, D), lambda i: (i, 0)),
            pl.BlockSpec((BN, 1), lambda i: (i, 0)),
        ],
        out_shape=[
            jax.ShapeDtypeStruct((NP, D), jnp.float32),
            jax.ShapeDtypeStruct((NP, 1), jnp.float32),
        ],
    )(hist, x)


def _mid_body(acc_ref, d_ref, y_ref):
    a = acc_ref[...]
    merged = jnp.concatenate([a[0], a[1]], axis=1)
    d = d_ref[...]
    y_ref[...] = merged * (d * d)


@jax.jit
def _tc_mid(acc, d):
    return pl.pallas_call(
        _mid_body,
        grid=(GRID_N,),
        in_specs=[
            pl.BlockSpec((NC, BN, HD), lambda i: (0, i, 0)),
            pl.BlockSpec((BN, 1), lambda i: (i, 0)),
        ],
        out_specs=pl.BlockSpec((BN, D), lambda i: (i, 0)),
        out_shape=jax.ShapeDtypeStruct((NP, D), jnp.float32),
    )(acc, d)


def _final_body(x_ref, a1_ref, a2_ref, d_ref, o_ref):
    a1 = a1_ref[...]
    a2 = a2_ref[...]
    e1 = jnp.concatenate([a1[0], a1[1]], axis=1)
    e2 = jnp.concatenate([a2[0], a2[1]], axis=1)
    d = d_ref[...]
    o_ref[...] = (x_ref[...] + d * e1 + d * e2) * (1.0 / 3.0)


@jax.jit
def _tc_final(x, acc1, acc2, d):
    return pl.pallas_call(
        _final_body,
        grid=(GRID_N,),
        in_specs=[
            pl.BlockSpec((BN, D), lambda i: (i, 0)),
            pl.BlockSpec((NC, BN, HD), lambda i: (0, i, 0)),
            pl.BlockSpec((NC, BN, HD), lambda i: (0, i, 0)),
            pl.BlockSpec((BN, 1), lambda i: (i, 0)),
        ],
        out_specs=pl.BlockSpec((BN, D), lambda i: (i, 0)),
        out_shape=jax.ShapeDtypeStruct((NP, D), jnp.float32),
    )(x, acc1, acc2, d)


def kernel(x, edge_index):
    src = edge_index[0].astype(jnp.int32)
    dst = edge_index[1].astype(jnp.int32)
    srcp = jnp.concatenate([src, jnp.zeros((EPAD - E,), jnp.int32)])
    dstp = jnp.concatenate([dst, jnp.full((EPAD - E,), DUMP, jnp.int32)])
    zrows = jnp.zeros((128, HD), jnp.float32)

    xp = jnp.concatenate(
        [x, jnp.zeros((NP - N, D), jnp.float32)], axis=0)

    hist = _sc_hist(dstp)
    y1, d = _tc_prep(hist, xp)
    acc1 = _sc_conv(y1.reshape(2 * NP, HD), srcp, dstp, zrows)
    y2 = _tc_mid(acc1, d)
    acc2 = _sc_conv(y2.reshape(2 * NP, HD), srcp, dstp, zrows)
    return _tc_final(xp, acc1, acc2, d)[:N]


# speedup vs baseline: 14.0411x; 1.1107x over previous
import jax
import jax.numpy as jnp
from jax import lax
from jax.experimental import pallas as pl
from jax.experimental.pallas import tpu as pltpu
from jax.experimental.pallas import tpu_sc as plsc

N = 50000
E = 800000
D = 64
HD = D // 2

NC, NS = 2, 16
NW = NC * NS

ACC_ROWS = 51200
DUMP = N
EPAD = 16 * ACC_ROWS
EW = EPAD // NS
HW = EPAD // NW
CH = 1024
CC = 256
SUB = CC // 128
NCH = EW // CC
ZR = ACC_ROWS // NS
NCHUNK_HIST = HW // CH

NP = ACC_ROWS
BN = 2048
GRID_N = NP // BN

import functools


@functools.lru_cache(maxsize=1)
def _mesh():
    return plsc.VectorSubcoreMesh(core_axis_name="c", subcore_axis_name="s")


_SC_PARAMS = pltpu.CompilerParams(needs_layout_passes=False,
                                 use_tc_tiling_on_sc=False)


def _hist_body(dst_hbm, hist_hbm, dv, hist_v):
    k = lax.axis_index("c")
    s = lax.axis_index("s")
    wid = k * NS + s

    @pl.loop(0, ACC_ROWS, step=16)
    def _(i):
        hist_v[pl.ds(i, 16)] = jnp.zeros((16,), jnp.float32)

    base = wid * HW

    @pl.loop(0, NCHUNK_HIST)
    def _(c):
        pltpu.sync_copy(dst_hbm.at[pl.ds(base + c * CH, CH)], dv)

        @pl.loop(0, CH, step=16)
        def _(i):
            plsc.addupdate_scatter(hist_v, [dv[pl.ds(i, 16)]],
                                   jnp.ones((16,), jnp.float32))

    pltpu.sync_copy(hist_v, hist_hbm.at[wid])


@jax.jit
def _sc_hist(dstp):
    kern = pl.kernel(
        _hist_body,
        out_type=jax.ShapeDtypeStruct((NW, ACC_ROWS), jnp.float32),
        mesh=_mesh(),
        scratch_types=[
            pltpu.VMEM((CH,), jnp.int32),
            pltpu.VMEM((ACC_ROWS,), jnp.float32),
        ],
        compiler_params=_SC_PARAMS,
    )
    return kern(dstp)


def _conv_body(y_hbm, gsrc_hbm, dst3_hbm, zero_hbm, acc_hbm,
               gvA, gvB, dvA, dvB, dvC, dvD, rowsA, rowsB,
               lsem, gsem, ssem, acc_sh):
    k = lax.axis_index("c")
    s = lax.axis_index("s")
    base = s * EW
    gsrc_k = gsrc_hbm.at[k]
    dst_s = dst3_hbm.at[s]

    pltpu.sync_copy(zero_hbm, acc_sh.at[pl.ds(s * ZR, ZR)])
    plsc.subcore_barrier()

    pltpu.async_copy(gsrc_k.at[pl.ds(base, CC)], gvA, lsem)
    pltpu.async_copy(dst_s.at[pl.ds(0, SUB)], dvA, lsem)

    gvs = (gvA, gvB)
    dvs = (dvA, dvB, dvC, dvD)
    rws = (rowsA, rowsB)

    def do_chunk(c, gv, dv, rows, gvn, dvn, drain, guard_prefetch):
        pltpu.make_async_copy(gsrc_k.at[pl.ds(0, CC)], gv, lsem).wait()
        pltpu.make_async_copy(dst_s.at[pl.ds(0, SUB)], dv, lsem).wait()

        def prefetch():
            nb = base + (c + 1) * CC
            pltpu.async_copy(gsrc_k.at[pl.ds(nb, CC)], gvn, lsem)
            pltpu.async_copy(dst_s.at[pl.ds((c + 1) * SUB, SUB)], dvn, lsem)
        if guard_prefetch is None:
            prefetch()
        else:
            pl.when(guard_prefetch)(prefetch)

        if drain:
            pltpu.make_async_copy(y_hbm.at[pl.ds(0, CC)], rows, ssem).wait()

        cps = [pltpu.async_copy(y_hbm.at[gv.at[pl.ds(j * 128, 128)]],
                                rows.at[pl.ds(j * 128, 128)], gsem)
               for j in range(SUB)]
        for cp in cps:
            cp.wait()
        for j in range(SUB):
            pltpu.async_copy(rows.at[pl.ds(j * 128, 128)],
                             acc_sh.at[dv.at[j]], ssem, add=True)

    for c in range(4):
        do_chunk(c, gvs[c % 2], dvs[c % 4], rws[c % 2],
                 gvs[(c + 1) % 2], dvs[(c + 1) % 4],
                 drain=(c >= 2), guard_prefetch=None)

    @pl.loop(0, (NCH - 4) // 4)
    def _(t):
        for u in range(4):
            c = 4 + t * 4 + u
            guard = (c + 1 < NCH) if u == 3 else None
            do_chunk(c, gvs[u % 2], dvs[u % 4], rws[u % 2],
                     gvs[(u + 1) % 2], dvs[(u + 1) % 4],
                     drain=True, guard_prefetch=guard)

    pltpu.make_async_copy(y_hbm.at[pl.ds(0, CC)], rowsA, ssem).wait()
    pltpu.make_async_copy(y_hbm.at[pl.ds(0, CC)], rowsB, ssem).wait()
    plsc.subcore_barrier()

    pltpu.sync_copy(acc_sh.at[pl.ds(s * ZR, ZR)],
                    acc_hbm.at[k].at[pl.ds(s * ZR, ZR)])


@jax.jit
def _sc_conv(yv, gsrc, dst3, zblk):
    kern = pl.kernel(
        _conv_body,
        out_type=jax.ShapeDtypeStruct((NC, ACC_ROWS, HD), jnp.float32),
        mesh=_mesh(),
        scratch_types=[
            pltpu.VMEM((CC,), jnp.int32),
            pltpu.VMEM((CC,), jnp.int32),
            pltpu.VMEM((SUB, 128), jnp.int32),
            pltpu.VMEM((SUB, 128), jnp.int32),
            pltpu.VMEM((SUB, 128), jnp.int32),
            pltpu.VMEM((SUB, 128), jnp.int32),
            pltpu.VMEM((CC, HD), jnp.float32),
            pltpu.VMEM((CC, HD), jnp.float32),
            pltpu.SemaphoreType.DMA,
            pltpu.SemaphoreType.DMA,
            pltpu.SemaphoreType.DMA,
            pltpu.VMEM_SHARED((ACC_ROWS, HD), jnp.float32),
        ],
        compiler_params=_SC_PARAMS,
    )
    return kern(yv, gsrc, dst3, zblk)


def _prep_body(hist_ref, x_ref, src_ref, y_ref, d_ref, g_ref):
    h = hist_ref[...]
    deg = jnp.sum(h, axis=0, keepdims=True)
    dinv = jnp.where(deg > 0, lax.rsqrt(deg), 0.0)
    dcol = jnp.transpose(dinv, (1, 0))
    d_ref[...] = dcol
    y_ref[...] = x_ref[...] * dcol
    g2 = src_ref[...] * 2
    g_ref[...] = jnp.concatenate([g2, g2 + 1], axis=0)


SB = (EPAD // 128) // GRID_N


@jax.jit
def _tc_prep(hist, x, src2d):
    return pl.pallas_call(
        _prep_body,
        grid=(GRID_N,),
        in_specs=[
            pl.BlockSpec((NW, BN), lambda i: (0, i)),
            pl.BlockSpec((BN, D), lambda i: (i, 0)),
            pl.BlockSpec((1, SB, 128), lambda i: (0, i, 0)),
        ],
        out_specs=[
            pl.BlockSpec((BN, D), lambda i: (i, 0)),
            pl.BlockSpec((BN, 1), lambda i: (i, 0)),
            pl.BlockSpec((NC, SB, 128), lambda i: (0, i, 0)),
        ],
        out_shape=[
            jax.ShapeDtypeStruct((NP, D), jnp.float32),
            jax.ShapeDtypeStruct((NP, 1), jnp.float32),
            jax.ShapeDtypeStruct((NC, EPAD // 128, 128), jnp.int32),
        ],
    )(hist, x, src2d)


def _mid_body(acc_ref, d_ref, y_ref):
    a = acc_ref[...]
    merged = jnp.concatenate([a[0], a[1]], axis=1)
    d = d_ref[...]
    y_ref[...] = merged * (d * d)


@jax.jit
def _tc_mid(acc, d):
    return pl.pallas_call(
        _mid_body,
        grid=(GRID_N,),
        in_specs=[
            pl.BlockSpec((NC, BN, HD), lambda i: (0, i, 0)),
            pl.BlockSpec((BN, 1), lambda i: (i, 0)),
        ],
        out_specs=pl.BlockSpec((BN, D), lambda i: (i, 0)),
        out_shape=jax.ShapeDtypeStruct((NP, D), jnp.float32),
    )(acc, d)


def _final_body(x_ref, a1_ref, a2_ref, d_ref, o_ref):
    a1 = a1_ref[...]
    a2 = a2_ref[...]
    e1 = jnp.concatenate([a1[0], a1[1]], axis=1)
    e2 = jnp.concatenate([a2[0], a2[1]], axis=1)
    d = d_ref[...]
    o_ref[...] = (x_ref[...] + d * e1 + d * e2) * (1.0 / 3.0)


@jax.jit
def _tc_final(x, acc1, acc2, d):
    return pl.pallas_call(
        _final_body,
        grid=(GRID_N,),
        in_specs=[
            pl.BlockSpec((BN, D), lambda i: (i, 0)),
            pl.BlockSpec((NC, BN, HD), lambda i: (0, i, 0)),
            pl.BlockSpec((NC, BN, HD), lambda i: (0, i, 0)),
            pl.BlockSpec((BN, 1), lambda i: (i, 0)),
        ],
        out_specs=pl.BlockSpec((BN, D), lambda i: (i, 0)),
        out_shape=jax.ShapeDtypeStruct((NP, D), jnp.float32),
    )(x, acc1, acc2, d)


def kernel(x, edge_index):
    src = edge_index[0].astype(jnp.int32)
    dst = edge_index[1].astype(jnp.int32)
    srcp = jnp.concatenate([src, jnp.zeros((EPAD - E,), jnp.int32)])
    dstp = jnp.concatenate([dst, jnp.full((EPAD - E,), DUMP, jnp.int32)])
    src2d = srcp.reshape(1, EPAD // 128, 128)
    dst3 = dstp.reshape(NS, EW // 128, 128)
    zblk = jnp.zeros((ZR, HD), jnp.float32)

    xp = jnp.concatenate(
        [x, jnp.zeros((NP - N, D), jnp.float32)], axis=0)

    hist = _sc_hist(dstp)
    y1, d, gsrc3 = _tc_prep(hist, xp, src2d)
    gsrc = gsrc3.reshape(NC, EPAD)
    acc1 = _sc_conv(y1.reshape(2 * NP, HD), gsrc, dst3, zblk)
    y2 = _tc_mid(acc1, d)
    acc2 = _sc_conv(y2.reshape(2 * NP, HD), gsrc, dst3, zblk)
    return _tc_final(xp, acc1, acc2, d)[:N]


# speedup vs baseline: 14.5818x; 1.0385x over previous
import jax
import jax.numpy as jnp
from jax import lax
from jax.experimental import pallas as pl
from jax.experimental.pallas import tpu as pltpu
from jax.experimental.pallas import tpu_sc as plsc

N = 50000
E = 800000
D = 64
HD = D // 2

NC, NS = 2, 16
NW = NC * NS

ACC_ROWS = 51200
DUMP = N
EPAD = 16 * ACC_ROWS
EW = EPAD // NS
HW = EPAD // NW
CH = 1024
CC = 256
SUB = CC // 128
NCH = EW // CC
ZR = ACC_ROWS // NS
NCHUNK_HIST = HW // CH

NP = ACC_ROWS
BN = 2048
GRID_N = NP // BN

import functools


@functools.lru_cache(maxsize=1)
def _mesh():
    return plsc.VectorSubcoreMesh(core_axis_name="c", subcore_axis_name="s")


_SC_PARAMS = pltpu.CompilerParams(needs_layout_passes=False,
                                 use_tc_tiling_on_sc=False)


def _hist_body(dst_hbm, hist_hbm, dv, hist_v):
    k = lax.axis_index("c")
    s = lax.axis_index("s")
    wid = k * NS + s

    @pl.loop(0, ACC_ROWS, step=16)
    def _(i):
        hist_v[pl.ds(i, 16)] = jnp.zeros((16,), jnp.float32)

    base = wid * HW

    @pl.loop(0, NCHUNK_HIST)
    def _(c):
        pltpu.sync_copy(dst_hbm.at[pl.ds(base + c * CH, CH)], dv)

        @pl.loop(0, CH, step=16)
        def _(i):
            plsc.addupdate_scatter(hist_v, [dv[pl.ds(i, 16)]],
                                   jnp.ones((16,), jnp.float32))

    pltpu.sync_copy(hist_v, hist_hbm.at[wid])


@jax.jit
def _sc_hist(dstp):
    kern = pl.kernel(
        _hist_body,
        out_type=jax.ShapeDtypeStruct((NW, ACC_ROWS), jnp.float32),
        mesh=_mesh(),
        scratch_types=[
            pltpu.VMEM((CH,), jnp.int32),
            pltpu.VMEM((ACC_ROWS,), jnp.float32),
        ],
        compiler_params=_SC_PARAMS,
    )
    return kern(dstp)


def _conv_body(y_hbm, gsrc_hbm, dst3_hbm, zero_hbm, acc_hbm,
               gvA, gvB, dvA, dvB, dvC, dvD, rowsA, rowsB,
               lsem, gsem, ssem, acc_sh):
    k = lax.axis_index("c")
    s = lax.axis_index("s")
    base = s * EW
    gsrc_k = gsrc_hbm.at[k]
    dst_s = dst3_hbm.at[s]

    pltpu.sync_copy(zero_hbm, acc_sh.at[pl.ds(s * ZR, ZR)])
    plsc.subcore_barrier()

    pltpu.async_copy(gsrc_k.at[pl.ds(base, CC)], gvA, lsem)
    pltpu.async_copy(dst_s.at[pl.ds(0, SUB)], dvA, lsem)

    gvs = (gvA, gvB)
    dvs = (dvA, dvB, dvC, dvD)
    rws = (rowsA, rowsB)

    def do_chunk(c, gv, dv, rows, gvn, dvn, drain, guard_prefetch):
        pltpu.make_async_copy(gsrc_k.at[pl.ds(0, CC)], gv, lsem).wait()
        pltpu.make_async_copy(dst_s.at[pl.ds(0, SUB)], dv, lsem).wait()

        def prefetch():
            nb = base + (c + 1) * CC
            pltpu.async_copy(gsrc_k.at[pl.ds(nb, CC)], gvn, lsem)
            pltpu.async_copy(dst_s.at[pl.ds((c + 1) * SUB, SUB)], dvn, lsem)
        if guard_prefetch is None:
            prefetch()
        else:
            pl.when(guard_prefetch)(prefetch)

        if drain:
            pltpu.make_async_copy(y_hbm.at[pl.ds(0, CC)], rows, ssem).wait()

        cps = [pltpu.async_copy(y_hbm.at[gv.at[pl.ds(j * 128, 128)]],
                                rows.at[pl.ds(j * 128, 128)], gsem)
               for j in range(SUB)]
        for cp in cps:
            cp.wait()
        for j in range(SUB):
            pltpu.async_copy(rows.at[pl.ds(j * 128, 128)],
                             acc_sh.at[dv.at[j]], ssem, add=True)

    for c in range(4):
        do_chunk(c, gvs[c % 2], dvs[c % 4], rws[c % 2],
                 gvs[(c + 1) % 2], dvs[(c + 1) % 4],
                 drain=(c >= 2), guard_prefetch=None)

    @pl.loop(0, (NCH - 4) // 4)
    def _(t):
        for u in range(4):
            c = 4 + t * 4 + u
            guard = (c + 1 < NCH) if u == 3 else None
            do_chunk(c, gvs[u % 2], dvs[u % 4], rws[u % 2],
                     gvs[(u + 1) % 2], dvs[(u + 1) % 4],
                     drain=True, guard_prefetch=guard)

    pltpu.make_async_copy(y_hbm.at[pl.ds(0, CC)], rowsA, ssem).wait()
    pltpu.make_async_copy(y_hbm.at[pl.ds(0, CC)], rowsB, ssem).wait()
    plsc.subcore_barrier()

    pltpu.sync_copy(acc_sh.at[pl.ds(s * ZR, ZR)],
                    acc_hbm.at[k].at[pl.ds(s * ZR, ZR)])


@jax.jit
def _sc_conv(yv, gsrc, dst3, zblk):
    kern = pl.kernel(
        _conv_body,
        out_type=jax.ShapeDtypeStruct((NC, ACC_ROWS, HD), jnp.float32),
        mesh=_mesh(),
        scratch_types=[
            pltpu.VMEM((CC,), jnp.int32),
            pltpu.VMEM((CC,), jnp.int32),
            pltpu.VMEM((SUB, 128), jnp.int32),
            pltpu.VMEM((SUB, 128), jnp.int32),
            pltpu.VMEM((SUB, 128), jnp.int32),
            pltpu.VMEM((SUB, 128), jnp.int32),
            pltpu.VMEM((CC, HD), jnp.float32),
            pltpu.VMEM((CC, HD), jnp.float32),
            pltpu.SemaphoreType.DMA,
            pltpu.SemaphoreType.DMA,
            pltpu.SemaphoreType.DMA,
            pltpu.VMEM_SHARED((ACC_ROWS, HD), jnp.float32),
        ],
        compiler_params=_SC_PARAMS,
    )
    return kern(yv, gsrc, dst3, zblk)


def _prep_body(hist_ref, x_ref, src_ref, y_ref, d_ref, g_ref):
    h = hist_ref[...]
    deg = jnp.sum(h, axis=0, keepdims=True)
    dinv = jnp.where(deg > 0, lax.rsqrt(deg), 0.0)
    dcol = jnp.transpose(dinv, (1, 0))
    d_ref[...] = dcol
    x = x_ref[...]
    odd = pl.program_id(0) % 2 == 1
    half = jnp.where(odd, x[:, HD:], x[:, :HD])
    y_ref[...] = half * dcol
    src = src_ref[...]
    ga = (src & ~(BN - 1)) * 2 + (src & (BN - 1))
    g_ref[...] = jnp.concatenate([ga, ga + BN], axis=0)


SB = (EPAD // 128) // GRID_N


@jax.jit
def _tc_prep(hist, xp, src2d):
    return pl.pallas_call(
        _prep_body,
        grid=(2 * GRID_N,),
        in_specs=[
            pl.BlockSpec((NW, BN), lambda j: (0, j // 2)),
            pl.BlockSpec((BN, D), lambda j: (j // 2, 0)),
            pl.BlockSpec((1, SB, 128), lambda j: (0, j // 2, 0)),
        ],
        out_specs=[
            pl.BlockSpec((BN, HD), lambda j: (j, 0)),
            pl.BlockSpec((BN, 1), lambda j: (j // 2, 0)),
            pl.BlockSpec((NC, SB, 128), lambda j: (0, j // 2, 0)),
        ],
        out_shape=[
            jax.ShapeDtypeStruct((2 * NP, HD), jnp.float32),
            jax.ShapeDtypeStruct((NP, 1), jnp.float32),
            jax.ShapeDtypeStruct((NC, EPAD // 128, 128), jnp.int32),
        ],
    )(hist, xp, src2d)


def _mid_body(acc_ref, d_ref, y_ref):
    a = acc_ref[...][0]
    d = d_ref[...]
    y_ref[...] = a * (d * d)


@jax.jit
def _tc_mid(acc, d):
    return pl.pallas_call(
        _mid_body,
        grid=(2 * GRID_N,),
        in_specs=[
            pl.BlockSpec((1, BN, HD), lambda j: (j % 2, j // 2, 0)),
            pl.BlockSpec((BN, 1), lambda j: (j // 2, 0)),
        ],
        out_specs=pl.BlockSpec((BN, HD), lambda j: (j, 0)),
        out_shape=jax.ShapeDtypeStruct((2 * NP, HD), jnp.float32),
    )(acc, d)


def _final_body(x_ref, a1_ref, a2_ref, d_ref, o_ref):
    a1 = a1_ref[...]
    a2 = a2_ref[...]
    e1 = jnp.concatenate([a1[0], a1[1]], axis=1)
    e2 = jnp.concatenate([a2[0], a2[1]], axis=1)
    d = d_ref[...]
    o_ref[...] = (x_ref[...] + d * e1 + d * e2) * (1.0 / 3.0)


BN2 = 2000


@jax.jit
def _tc_final(x, acc1, acc2, d):
    return pl.pallas_call(
        _final_body,
        grid=(N // BN2,),
        in_specs=[
            pl.BlockSpec((BN2, D), lambda i: (i, 0)),
            pl.BlockSpec((NC, BN2, HD), lambda i: (0, i, 0)),
            pl.BlockSpec((NC, BN2, HD), lambda i: (0, i, 0)),
            pl.BlockSpec((BN2, 1), lambda i: (i, 0)),
        ],
        out_specs=pl.BlockSpec((BN2, D), lambda i: (i, 0)),
        out_shape=jax.ShapeDtypeStruct((N, D), jnp.float32),
    )(x, acc1, acc2, d)


def kernel(x, edge_index):
    src = edge_index[0].astype(jnp.int32)
    dst = edge_index[1].astype(jnp.int32)
    srcp = jnp.concatenate([src, jnp.zeros((EPAD - E,), jnp.int32)])
    dstp = jnp.concatenate([dst, jnp.full((EPAD - E,), DUMP, jnp.int32)])
    src2d = srcp.reshape(1, EPAD // 128, 128)
    dst3 = dstp.reshape(NS, EW // 128, 128)
    zblk = jnp.zeros((ZR, HD), jnp.float32)

    xp = jnp.concatenate(
        [x, jnp.zeros((NP - N, D), jnp.float32)], axis=0)
    hist = _sc_hist(dstp)
    y1, d, gsrc3 = _tc_prep(hist, xp, src2d)
    gsrc = gsrc3.reshape(NC, EPAD)
    acc1 = _sc_conv(y1, gsrc, dst3, zblk)
    y2 = _tc_mid(acc1, d)
    acc2 = _sc_conv(y2, gsrc, dst3, zblk)
    return _tc_final(x, acc1, acc2, d)


# speedup vs baseline: 15.1025x; 1.0357x over previous
import jax
import jax.numpy as jnp
from jax import lax
from jax.experimental import pallas as pl
from jax.experimental.pallas import tpu as pltpu
from jax.experimental.pallas import tpu_sc as plsc

N = 50000
E = 800000
D = 64
HD = D // 2

NC, NS = 2, 16
NW = NC * NS

ACC_ROWS = 51200
DUMP = N
EPAD = 16 * ACC_ROWS
EW = EPAD // NS
HW = EPAD // NW
CH = 1024
CC = 256
SUB = CC // 128
NCH = EW // CC
ZR = ACC_ROWS // NS
NG = EPAD // CC
NCHUNK_HIST = HW // CH

NP = ACC_ROWS
BN = 2048
GRID_N = NP // BN

import functools


@functools.lru_cache(maxsize=1)
def _mesh():
    return plsc.VectorSubcoreMesh(core_axis_name="c", subcore_axis_name="s")


_SC_PARAMS = pltpu.CompilerParams(needs_layout_passes=False,
                                 use_tc_tiling_on_sc=False)


def _hist_body(dst_hbm, hist_hbm, dv, hist_v):
    k = lax.axis_index("c")
    s = lax.axis_index("s")
    wid = k * NS + s

    @pl.loop(0, ACC_ROWS, step=16)
    def _(i):
        hist_v[pl.ds(i, 16)] = jnp.zeros((16,), jnp.float32)

    base = wid * HW

    @pl.loop(0, NCHUNK_HIST)
    def _(c):
        pltpu.sync_copy(dst_hbm.at[pl.ds(base + c * CH, CH)], dv)

        @pl.loop(0, CH, step=16)
        def _(i):
            plsc.addupdate_scatter(hist_v, [dv[pl.ds(i, 16)]],
                                   jnp.ones((16,), jnp.float32))

    pltpu.sync_copy(hist_v, hist_hbm.at[wid])


@jax.jit
def _sc_hist(dstp):
    kern = pl.kernel(
        _hist_body,
        out_type=jax.ShapeDtypeStruct((NW, ACC_ROWS), jnp.float32),
        mesh=_mesh(),
        scratch_types=[
            pltpu.VMEM((CH,), jnp.int32),
            pltpu.VMEM((ACC_ROWS,), jnp.float32),
        ],
        compiler_params=_SC_PARAMS,
    )
    return kern(dstp)


def _conv_body(y_hbm, pk_hbm, zero_hbm, acc_hbm,
               pvA, pvB, pvC, pvD, rowsA, rowsB, lsem, gsem, ssem, acc_sh):
    k = lax.axis_index("c")
    s = lax.axis_index("s")
    gbase = s * NCH
    pk_k = pk_hbm.at[k]

    pltpu.sync_copy(zero_hbm, acc_sh.at[pl.ds(s * ZR, ZR)])
    plsc.subcore_barrier()

    pvs = (pvA, pvB, pvC, pvD)
    rws = (rowsA, rowsB)

    pltpu.async_copy(pk_k.at[pl.ds(gbase * 4, 4)], pvA, lsem)

    def do_chunk(c, m4, drain, guard):
        pv, rows = pvs[m4], rws[m4 % 2]
        pltpu.make_async_copy(pk_k.at[pl.ds(0, 4)], pv, lsem).wait()

        def prefetch():
            pltpu.async_copy(pk_k.at[pl.ds((gbase + c + 1) * 4, 4)],
                             pvs[(m4 + 1) % 4], lsem)
        if guard is None:
            prefetch()
        else:
            pl.when(guard)(prefetch)

        if drain:
            pltpu.make_async_copy(y_hbm.at[pl.ds(0, CC)], rows, ssem).wait()

        cps = [pltpu.async_copy(y_hbm.at[pv.at[j]],
                                rows.at[pl.ds(j * 128, 128)], gsem)
               for j in range(SUB)]
        for cp in cps:
            cp.wait()
        for j in range(SUB):
            pltpu.async_copy(rows.at[pl.ds(j * 128, 128)],
                             acc_sh.at[pv.at[SUB + j]], ssem, add=True)

    for c in range(4):
        do_chunk(c, c, drain=(c >= 2), guard=None)

    @pl.loop(0, (NCH - 4) // 4)
    def _(t):
        for u in range(4):
            c = 4 + t * 4 + u
            do_chunk(c, u, drain=True,
                     guard=(c + 1 < NCH) if u == 3 else None)

    pltpu.make_async_copy(y_hbm.at[pl.ds(0, CC)], rowsA, ssem).wait()
    pltpu.make_async_copy(y_hbm.at[pl.ds(0, CC)], rowsB, ssem).wait()
    plsc.subcore_barrier()

    pltpu.sync_copy(acc_sh.at[pl.ds(s * ZR, ZR)],
                    acc_hbm.at[k].at[pl.ds(s * ZR, ZR)])


@jax.jit
def _sc_conv(yv, pk, zblk):
    kern = pl.kernel(
        _conv_body,
        out_type=jax.ShapeDtypeStruct((NC, ACC_ROWS, HD), jnp.float32),
        mesh=_mesh(),
        scratch_types=[
            pltpu.VMEM((4, 128), jnp.int32),
            pltpu.VMEM((4, 128), jnp.int32),
            pltpu.VMEM((4, 128), jnp.int32),
            pltpu.VMEM((4, 128), jnp.int32),
            pltpu.VMEM((CC, HD), jnp.float32),
            pltpu.VMEM((CC, HD), jnp.float32),
            pltpu.SemaphoreType.DMA,
            pltpu.SemaphoreType.DMA,
            pltpu.SemaphoreType.DMA,
            pltpu.VMEM_SHARED((ACC_ROWS, HD), jnp.float32),
        ],
        compiler_params=_SC_PARAMS,
    )
    return kern(yv, pk, zblk)


def _prep_body(hist_ref, x_ref, src_ref, dst_ref, y_ref, d_ref, g_ref):
    h = hist_ref[...]
    deg = jnp.sum(h, axis=0, keepdims=True)
    dinv = jnp.where(deg > 0, lax.rsqrt(deg), 0.0)
    dcol = jnp.transpose(dinv, (1, 0))
    d_ref[...] = dcol
    x = x_ref[...]
    odd = pl.program_id(0) % 2 == 1
    half = jnp.where(odd, x[:, HD:], x[:, :HD])
    y_ref[...] = half * dcol
    src = src_ref[...][0]
    dst = dst_ref[...][0]
    ga = (src & ~(BN - 1)) * 2 + (src & (BN - 1))
    a = ga.reshape(SB // 2, 2, 128)
    b = dst.reshape(SB // 2, 2, 128)
    pk = jnp.stack([jnp.concatenate([a, b], axis=1),
                    jnp.concatenate([a + BN, b], axis=1)], axis=0)
    g_ref[...] = pk.reshape(NC, 2 * SB, 128)


SB = (EPAD // 128) // GRID_N


@jax.jit
def _tc_prep(hist, xp, src2d, dst2d):
    return pl.pallas_call(
        _prep_body,
        grid=(2 * GRID_N,),
        in_specs=[
            pl.BlockSpec((NW, BN), lambda j: (0, j // 2)),
            pl.BlockSpec((BN, D), lambda j: (j // 2, 0)),
            pl.BlockSpec((1, SB, 128), lambda j: (0, j // 2, 0)),
            pl.BlockSpec((1, SB, 128), lambda j: (0, j // 2, 0)),
        ],
        out_specs=[
            pl.BlockSpec((BN, HD), lambda j: (j, 0)),
            pl.BlockSpec((BN, 1), lambda j: (j // 2, 0)),
            pl.BlockSpec((NC, 2 * SB, 128), lambda j: (0, j // 2, 0)),
        ],
        out_shape=[
            jax.ShapeDtypeStruct((2 * NP, HD), jnp.float32),
            jax.ShapeDtypeStruct((NP, 1), jnp.float32),
            jax.ShapeDtypeStruct((NC, 4 * NG, 128), jnp.int32),
        ],
    )(hist, xp, src2d, dst2d)


def _mid_body(acc_ref, d_ref, y_ref):
    a = acc_ref[...][0]
    d = d_ref[...]
    y_ref[...] = a * (d * d)


@jax.jit
def _tc_mid(acc, d):
    return pl.pallas_call(
        _mid_body,
        grid=(2 * GRID_N,),
        in_specs=[
            pl.BlockSpec((1, BN, HD), lambda j: (j % 2, j // 2, 0)),
            pl.BlockSpec((BN, 1), lambda j: (j // 2, 0)),
        ],
        out_specs=pl.BlockSpec((BN, HD), lambda j: (j, 0)),
        out_shape=jax.ShapeDtypeStruct((2 * NP, HD), jnp.float32),
    )(acc, d)


def _final_body(x_ref, a1_ref, a2_ref, d_ref, o_ref):
    a1 = a1_ref[...]
    a2 = a2_ref[...]
    e1 = jnp.concatenate([a1[0], a1[1]], axis=1)
    e2 = jnp.concatenate([a2[0], a2[1]], axis=1)
    d = d_ref[...]
    o_ref[...] = (x_ref[...] + d * e1 + d * e2) * (1.0 / 3.0)


BN2 = 2000


@jax.jit
def _tc_final(x, acc1, acc2, d):
    return pl.pallas_call(
        _final_body,
        grid=(N // BN2,),
        in_specs=[
            pl.BlockSpec((BN2, D), lambda i: (i, 0)),
            pl.BlockSpec((NC, BN2, HD), lambda i: (0, i, 0)),
            pl.BlockSpec((NC, BN2, HD), lambda i: (0, i, 0)),
            pl.BlockSpec((BN2, 1), lambda i: (i, 0)),
        ],
        out_specs=pl.BlockSpec((BN2, D), lambda i: (i, 0)),
        out_shape=jax.ShapeDtypeStruct((N, D), jnp.float32),
    )(x, acc1, acc2, d)


def kernel(x, edge_index):
    src = edge_index[0].astype(jnp.int32)
    dst = edge_index[1].astype(jnp.int32)
    srcp = jnp.concatenate([src, jnp.zeros((EPAD - E,), jnp.int32)])
    dstp = jnp.concatenate([dst, jnp.full((EPAD - E,), DUMP, jnp.int32)])
    src2d = srcp.reshape(1, EPAD // 128, 128)
    dst2d = dstp.reshape(1, EPAD // 128, 128)
    zblk = jnp.zeros((ZR, HD), jnp.float32)

    xp = jnp.concatenate(
        [x, jnp.zeros((NP - N, D), jnp.float32)], axis=0)
    hist = _sc_hist(dstp)
    y1, d, pk = _tc_prep(hist, xp, src2d, dst2d)
    acc1 = _sc_conv(y1, pk, zblk)
    y2 = _tc_mid(acc1, d)
    acc2 = _sc_conv(y2, pk, zblk)
    return _tc_final(x, acc1, acc2, d)
